# Initial kernel scaffold; baseline (speedup 1.0000x reference)
import functools

import jax
import jax.numpy as jnp
from jax import lax
from jax.experimental import pallas as pl
from jax.experimental.pallas import tpu as pltpu
from jax.experimental.pallas import tpu_sc as plsc

N = 10000
E = 320000
D = 128
L = 5
G = 64
S = 512

NC = 2
NS = 16
NW = NC * NS

EDGE_CHUNK = 128
N_EDGE_CHUNKS = E // EDGE_CHUNK
ROWS_PER_SUBCORE = N // NS

POOL_CHUNK = 80
N_POOL_CHUNKS = N // POOL_CHUNK
POOL_ROWS_PER_SUBCORE = S // NS

_mesh = plsc.VectorSubcoreMesh(core_axis_name="c", subcore_axis_name="s")


def _zero_vmem_2d(ref, rows, cols):
    z = jnp.zeros((16,), jnp.float32)

    @pl.loop(0, rows)
    def _(r):
        @pl.loop(0, cols // 16)
        def _(c):
            ref[r, pl.ds(c * 16, 16)] = z


@functools.partial(
    pl.kernel,
    out_type=jax.ShapeDtypeStruct((NC, N, D), jnp.float32),
    mesh=_mesh,
    scratch_types=[
        pltpu.VMEM((EDGE_CHUNK,), jnp.int32),
        pltpu.VMEM((EDGE_CHUNK,), jnp.int32),
        pltpu.VMEM((EDGE_CHUNK, D), jnp.float32),
        pltpu.VMEM_SHARED((N, D), jnp.float32),
        pltpu.SemaphoreType.DMA,
    ],
)
def _sc_segment_sum(h_hbm, src_hbm, dst_hbm, out_hbm,
                    src_v, dst_v, rows_v, acc_sh, sem):
    cid = lax.axis_index("c")
    sid = lax.axis_index("s")
    wid = sid * NC + cid

    _zero_vmem_2d(rows_v, EDGE_CHUNK, D)
    base = sid * ROWS_PER_SUBCORE
    for k in range(4):
        pltpu.sync_copy(rows_v,
                        acc_sh.at[pl.ds(base + k * EDGE_CHUNK, EDGE_CHUNK)])
    pltpu.sync_copy(rows_v.at[pl.ds(0, ROWS_PER_SUBCORE - 4 * EDGE_CHUNK)],
                    acc_sh.at[pl.ds(base + 4 * EDGE_CHUNK,
                                    ROWS_PER_SUBCORE - 4 * EDGE_CHUNK)])
    plsc.subcore_barrier()

    @pl.loop(wid, N_EDGE_CHUNKS, step=NW)
    def _(i):
        ebase = i * EDGE_CHUNK
        pltpu.sync_copy(src_hbm.at[pl.ds(ebase, EDGE_CHUNK)], src_v)
        pltpu.sync_copy(dst_hbm.at[pl.ds(ebase, EDGE_CHUNK)], dst_v)
        pltpu.async_copy(h_hbm.at[src_v], rows_v, sem).wait()
        pltpu.sync_copy(rows_v, acc_sh.at[dst_v], add=True)

    plsc.subcore_barrier()
    for k in range(4):
        pltpu.sync_copy(acc_sh.at[pl.ds(base + k * EDGE_CHUNK, EDGE_CHUNK)],
                        out_hbm.at[cid, pl.ds(base + k * EDGE_CHUNK, EDGE_CHUNK)])
    pltpu.sync_copy(acc_sh.at[pl.ds(base + 4 * EDGE_CHUNK,
                                    ROWS_PER_SUBCORE - 4 * EDGE_CHUNK)],
                    out_hbm.at[cid, pl.ds(base + 4 * EDGE_CHUNK,
                                          ROWS_PER_SUBCORE - 4 * EDGE_CHUNK)])


def _tc_layer_body(h_ref, p_ref, wrel_ref, wroot_ref, brel_ref,
                   gamma_ref, beta_ref, o_ref):
    agg = p_ref[0] + p_ref[1]
    out = (jnp.dot(agg, wrel_ref[...], preferred_element_type=jnp.float32)
           + jnp.dot(h_ref[...], wroot_ref[...],
                     preferred_element_type=jnp.float32)
           + brel_ref[...])
    mu = jnp.mean(out, axis=0, keepdims=True)
    var = jnp.mean((out - mu) ** 2, axis=0, keepdims=True)
    normed = (out - mu) * lax.rsqrt(var + 1e-5) * gamma_ref[...] + beta_ref[...]
    o_ref[...] = jnp.maximum(normed, 0.0)


_tc_layer = pl.pallas_call(
    _tc_layer_body,
    out_shape=jax.ShapeDtypeStruct((N, D), jnp.float32),
)


@functools.partial(
    pl.kernel,
    out_type=[jax.ShapeDtypeStruct((NC, S, D), jnp.float32),
              jax.ShapeDtypeStruct((NC, S, 16), jnp.float32)],
    mesh=_mesh,
    scratch_types=[
        pltpu.VMEM((G,), jnp.int32),
        pltpu.VMEM((G,), jnp.int32),
        pltpu.VMEM((POOL_CHUNK,), jnp.int32),
        pltpu.VMEM((POOL_CHUNK,), jnp.int32),
        pltpu.VMEM((POOL_CHUNK,), jnp.int32),
        pltpu.VMEM((POOL_CHUNK, D), jnp.float32),
        pltpu.VMEM((POOL_CHUNK, 16), jnp.float32),
        pltpu.VMEM((POOL_ROWS_PER_SUBCORE, 16), jnp.float32),
        pltpu.VMEM_SHARED((S, D), jnp.float32),
        pltpu.VMEM_SHARED((S, 16), jnp.float32),
    ],
)
def _sc_pool(h_hbm, batch_hbm, sb_hbm, ns_hbm, sum_hbm, cnt_hbm,
             ns_v, offs_v, bt_v, sb_v, id_v, rows_v, ones_v, zc_v,
             acc_sh, cnt_sh):
    cid = lax.axis_index("c")
    sid = lax.axis_index("s")
    wid = sid * NC + cid

    pltpu.sync_copy(ns_hbm, ns_v)
    carry = jnp.int32(0)
    for k in range(G // 16):
        v = ns_v[pl.ds(k * 16, 16)]
        incl = plsc.cumsum(v)
        offs_v[pl.ds(k * 16, 16)] = incl - v + carry
        carry = carry + jnp.sum(v)

    _zero_vmem_2d(rows_v, POOL_CHUNK, D)
    one = jnp.ones((16,), jnp.float32)

    @pl.loop(0, POOL_CHUNK)
    def _(r):
        ones_v[r, pl.ds(0, 16)] = one

    @pl.loop(0, POOL_ROWS_PER_SUBCORE)
    def _(r):
        zc_v[r, pl.ds(0, 16)] = jnp.zeros((16,), jnp.float32)

    pbase = sid * POOL_ROWS_PER_SUBCORE
    pltpu.sync_copy(rows_v.at[pl.ds(0, POOL_ROWS_PER_SUBCORE)],
                    acc_sh.at[pl.ds(pbase, POOL_ROWS_PER_SUBCORE)])
    pltpu.sync_copy(zc_v, cnt_sh.at[pl.ds(pbase, POOL_ROWS_PER_SUBCORE)])
    plsc.subcore_barrier()

    @pl.loop(wid, N_POOL_CHUNKS, step=NW)
    def _(i):
        nbase = i * POOL_CHUNK
        pltpu.sync_copy(batch_hbm.at[pl.ds(nbase, POOL_CHUNK)], bt_v)
        pltpu.sync_copy(sb_hbm.at[pl.ds(nbase, POOL_CHUNK)], sb_v)
        for k in range(POOL_CHUNK // 16):
            idx16 = bt_v[pl.ds(k * 16, 16)]
            off16 = plsc.load_gather(offs_v, [idx16])
            id_v[pl.ds(k * 16, 16)] = sb_v[pl.ds(k * 16, 16)] + off16
        pltpu.sync_copy(h_hbm.at[pl.ds(nbase, POOL_CHUNK)], rows_v)
        pltpu.sync_copy(rows_v, acc_sh.at[id_v], add=True)
        pltpu.sync_copy(ones_v, cnt_sh.at[id_v], add=True)

    plsc.subcore_barrier()
    pltpu.sync_copy(acc_sh.at[pl.ds(pbase, POOL_ROWS_PER_SUBCORE)],
                    sum_hbm.at[cid, pl.ds(pbase, POOL_ROWS_PER_SUBCORE)])
    pltpu.sync_copy(cnt_sh.at[pl.ds(pbase, POOL_ROWS_PER_SUBCORE)],
                    cnt_hbm.at[cid, pl.ds(pbase, POOL_ROWS_PER_SUBCORE)])


def _tc_finalize_body(s_ref, c_ref, o_ref):
    s = s_ref[0] + s_ref[1]
    c = c_ref[0] + c_ref[1]
    o_ref[...] = s / jnp.maximum(c[:, 0:1], 1.0)


_tc_finalize = pl.pallas_call(
    _tc_finalize_body,
    out_shape=jax.ShapeDtypeStruct((S, D), jnp.float32),
)


def kernel(x, edge_index, edge_attr, batch, num_subgraphs, subgraph_batch,
           Wroot, Wrel, brel, gamma, beta):
    src = edge_index[0]
    dst = edge_index[1]
    h = x
    for l in range(L):
        partials = _sc_segment_sum(h, src, dst)
        h = _tc_layer(h, partials, Wrel[l], Wroot[l],
                      brel[l].reshape(1, D), gamma[l].reshape(1, D),
                      beta[l].reshape(1, D))
    sums, cnts = _sc_pool(h, batch, subgraph_batch, num_subgraphs)
    return _tc_finalize(sums, cnts)


# speedup vs baseline: 5.6407x; 5.6407x over previous
import dataclasses
import functools

import jax
import jax.numpy as jnp
from jax import lax
from jax.experimental import pallas as pl
from jax.experimental.pallas import tpu as pltpu
from jax.experimental.pallas import tpu_sc as plsc

N = 10000
E = 320000
D = 128
L = 5
G = 64
S = 512

NC = 2
NS = 16
NW = NC * NS

EDGE_CHUNK = 128
N_EDGE_CHUNKS = E // EDGE_CHUNK
ROW_CHUNK = 80
N_ROW_CHUNKS = N // ROW_CHUNK

POOL_CHUNK = 80
N_POOL_CHUNKS = N // POOL_CHUNK
POOL_ROWS_PER_SUBCORE = S // NS

_mesh = plsc.VectorSubcoreMesh(core_axis_name="c", subcore_axis_name="s")

_sc_params = pltpu.CompilerParams()
if "needs_layout_passes" in pltpu.CompilerParams.__dataclass_fields__:
    _sc_params = dataclasses.replace(_sc_params, needs_layout_passes=False)


def _zero_vmem_2d(ref, rows, cols):
    z = jnp.zeros((16,), jnp.float32)

    @pl.loop(0, rows)
    def _(r):
        @pl.loop(0, cols // 16)
        def _(c):
            ref[r, pl.ds(c * 16, 16)] = z


@functools.partial(
    pl.kernel,
    out_type=jax.ShapeDtypeStruct((NC, N, D), jnp.float32),
    mesh=_mesh,
    scratch_types=[
        pltpu.VMEM((EDGE_CHUNK,), jnp.int32),
        pltpu.VMEM((EDGE_CHUNK,), jnp.int32),
        pltpu.VMEM((EDGE_CHUNK, D), jnp.float32),
        pltpu.VMEM_SHARED((N, D), jnp.float32),
        pltpu.SemaphoreType.DMA,
    ],
)
def _sc_segment_sum(h_hbm, src_hbm, dst_hbm, out_hbm,
                    src_v, dst_v, rows_v, acc_sh, sem):
    cid = lax.axis_index("c")
    sid = lax.axis_index("s")
    wid = sid * NC + cid

    _zero_vmem_2d(rows_v, ROW_CHUNK, D)

    @pl.loop(sid, N_ROW_CHUNKS, step=NS)
    def _(j):
        pltpu.sync_copy(rows_v.at[pl.ds(0, ROW_CHUNK)],
                        acc_sh.at[pl.ds(j * ROW_CHUNK, ROW_CHUNK)])

    plsc.subcore_barrier()

    @pl.loop(wid, N_EDGE_CHUNKS, step=NW)
    def _(i):
        ebase = i * EDGE_CHUNK
        pltpu.sync_copy(src_hbm.at[pl.ds(ebase, EDGE_CHUNK)], src_v)
        pltpu.sync_copy(dst_hbm.at[pl.ds(ebase, EDGE_CHUNK)], dst_v)
        pltpu.async_copy(h_hbm.at[src_v], rows_v, sem).wait()
        pltpu.sync_copy(rows_v, acc_sh.at[dst_v], add=True)

    plsc.subcore_barrier()

    @pl.loop(sid, N_ROW_CHUNKS, step=NS)
    def _(j):
        pltpu.sync_copy(acc_sh.at[pl.ds(j * ROW_CHUNK, ROW_CHUNK)],
                        out_hbm.at[cid, pl.ds(j * ROW_CHUNK, ROW_CHUNK)])


def _tc_layer_body(h_ref, p_ref, wrel_ref, wroot_ref, brel_ref,
                   gamma_ref, beta_ref, o_ref):
    agg = p_ref[0] + p_ref[1]
    out = (jnp.dot(agg, wrel_ref[...], preferred_element_type=jnp.float32)
           + jnp.dot(h_ref[...], wroot_ref[...],
                     preferred_element_type=jnp.float32)
           + brel_ref[...])
    mu = jnp.mean(out, axis=0, keepdims=True)
    var = jnp.mean((out - mu) ** 2, axis=0, keepdims=True)
    normed = (out - mu) * lax.rsqrt(var + 1e-5) * gamma_ref[...] + beta_ref[...]
    o_ref[...] = jnp.maximum(normed, 0.0)


_tc_layer = pl.pallas_call(
    _tc_layer_body,
    out_shape=jax.ShapeDtypeStruct((N, D), jnp.float32),
)


@functools.partial(
    pl.kernel,
    out_type=[jax.ShapeDtypeStruct((NC, S, D), jnp.float32),
              jax.ShapeDtypeStruct((NC, S, D), jnp.float32)],
    mesh=_mesh,
    scratch_types=[
        pltpu.VMEM((G,), jnp.int32),
        pltpu.VMEM((G,), jnp.int32),
        pltpu.VMEM((POOL_CHUNK,), jnp.int32),
        pltpu.VMEM((POOL_CHUNK,), jnp.int32),
        pltpu.VMEM((POOL_CHUNK,), jnp.int32),
        pltpu.VMEM((POOL_CHUNK, D), jnp.float32),
        pltpu.VMEM((POOL_CHUNK, D), jnp.float32),
        pltpu.VMEM_SHARED((S, D), jnp.float32),
        pltpu.VMEM_SHARED((S, D), jnp.float32),
    ],
    compiler_params=_sc_params,
)
def _sc_pool(h_hbm, batch_hbm, sb_hbm, ns_hbm, sum_hbm, cnt_hbm,
             ns_v, offs_v, bt_v, sb_v, id_v, rows_v, ones_v,
             acc_sh, cnt_sh):
    cid = lax.axis_index("c")
    sid = lax.axis_index("s")
    wid = sid * NC + cid

    pltpu.sync_copy(ns_hbm, ns_v)
    carry = jnp.int32(0)
    for k in range(G // 16):
        v = ns_v[pl.ds(k * 16, 16)]
        incl = plsc.cumsum(v)
        offs_v[pl.ds(k * 16, 16)] = incl - v + carry
        carry = carry + jnp.sum(v)

    _zero_vmem_2d(rows_v, POOL_CHUNK, D)
    one = jnp.ones((16,), jnp.float32)

    @pl.loop(0, POOL_CHUNK)
    def _(r):
        @pl.loop(0, D // 16)
        def _(c):
            ones_v[r, pl.ds(c * 16, 16)] = one

    pbase = sid * POOL_ROWS_PER_SUBCORE
    pltpu.sync_copy(rows_v.at[pl.ds(0, POOL_ROWS_PER_SUBCORE)],
                    acc_sh.at[pl.ds(pbase, POOL_ROWS_PER_SUBCORE)])
    pltpu.sync_copy(rows_v.at[pl.ds(0, POOL_ROWS_PER_SUBCORE)],
                    cnt_sh.at[pl.ds(pbase, POOL_ROWS_PER_SUBCORE)])
    plsc.subcore_barrier()

    @pl.loop(wid, N_POOL_CHUNKS, step=NW)
    def _(i):
        nbase = i * POOL_CHUNK
        pltpu.sync_copy(batch_hbm.at[pl.ds(nbase, POOL_CHUNK)], bt_v)
        pltpu.sync_copy(sb_hbm.at[pl.ds(nbase, POOL_CHUNK)], sb_v)
        for k in range(POOL_CHUNK // 16):
            idx16 = bt_v[pl.ds(k * 16, 16)]
            off16 = plsc.load_gather(offs_v, [idx16])
            id_v[pl.ds(k * 16, 16)] = sb_v[pl.ds(k * 16, 16)] + off16
        pltpu.sync_copy(h_hbm.at[pl.ds(nbase, POOL_CHUNK)], rows_v)
        pltpu.sync_copy(rows_v, acc_sh.at[id_v], add=True)
        pltpu.sync_copy(ones_v, cnt_sh.at[id_v], add=True)

    plsc.subcore_barrier()
    pltpu.sync_copy(acc_sh.at[pl.ds(pbase, POOL_ROWS_PER_SUBCORE)],
                    sum_hbm.at[cid, pl.ds(pbase, POOL_ROWS_PER_SUBCORE)])
    pltpu.sync_copy(cnt_sh.at[pl.ds(pbase, POOL_ROWS_PER_SUBCORE)],
                    cnt_hbm.at[cid, pl.ds(pbase, POOL_ROWS_PER_SUBCORE)])


def _tc_finalize_body(s_ref, c_ref, o_ref):
    s = s_ref[0] + s_ref[1]
    c = c_ref[0] + c_ref[1]
    o_ref[...] = s / jnp.maximum(c[:, 0:1], 1.0)


_tc_finalize = pl.pallas_call(
    _tc_finalize_body,
    out_shape=jax.ShapeDtypeStruct((S, D), jnp.float32),
)


def kernel(x, edge_index, edge_attr, batch, num_subgraphs, subgraph_batch,
           Wroot, Wrel, brel, gamma, beta):
    src = edge_index[0]
    dst = edge_index[1]
    h = x
    for l in range(L):
        partials = _sc_segment_sum(h, src, dst)
        h = _tc_layer(h, partials, Wrel[l], Wroot[l],
                      brel[l].reshape(1, D), gamma[l].reshape(1, D),
                      beta[l].reshape(1, D))
    sums, cnts = _sc_pool(h, batch, subgraph_batch, num_subgraphs)
    return _tc_finalize(sums, cnts)
